# baseline (speedup 1.0000x reference)
import functools

import jax
import jax.numpy as jnp
from jax import lax
from jax.experimental import pallas as pl
from jax.experimental.pallas import tpu as pltpu
from jax.experimental.pallas import tpu_sc as plsc

NC = 2
NS = 16
NW = NC * NS
RPB = 128
NBUF = 2


def _gather_fn(batch, seq, d):
    mesh = plsc.VectorSubcoreMesh(
        core_axis_name="c", subcore_axis_name="s",
        num_cores=NC, num_subcores=NS,
    )
    dd = 2 * d

    @functools.partial(
        pl.kernel,
        out_type=jax.ShapeDtypeStruct((seq, d, batch), jnp.float32),
        mesh=mesh,
        compiler_params=pltpu.CompilerParams(needs_layout_passes=False),
        scratch_types=[
            pltpu.VMEM((seq, RPB), jnp.int32),
            pltpu.VMEM((seq, RPB), jnp.int32),
            pltpu.VMEM((8, 16), jnp.int32),
            pltpu.VMEM((NBUF, RPB, dd), jnp.float32),
            pltpu.VMEM((NBUF, d, RPB), jnp.float32),
            pltpu.SemaphoreType.DMA,
            pltpu.SemaphoreType.DMA,
        ],
    )
    def gather_kernel(ids_hbm, table_hbm, out_hbm, idx_v, idx2_v, off_v,
                      gbuf, tbuf, gsem, ssem):
        wid = lax.axis_index("s") * NC + lax.axis_index("c")
        b0 = wid * RPB
        pltpu.sync_copy(ids_hbm.at[:, pl.ds(b0, RPB)], idx_v)

        @pl.loop(0, seq)
        def _(j):
            for k in range(RPB // 16):
                v = idx_v[j, pl.ds(16 * k, 16)]
                idx2_v[j, pl.ds(16 * k, 16)] = lax.shift_right_logical(v, 1)

        for b in range(NBUF):
            pltpu.async_copy(table_hbm.at[idx2_v.at[b]], gbuf.at[b], gsem)

        @pl.loop(0, seq, step=NBUF)
        def _(g):
            for b in range(NBUF):
                j = g + b
                pltpu.make_async_copy(
                    table_hbm.at[idx2_v.at[0]], gbuf.at[b], gsem
                ).wait()
                for k in range(RPB // 16):
                    v = idx_v[j, pl.ds(16 * k, 16)]
                    off_v[k, :] = lax.shift_left(
                        lax.bitwise_and(v, jnp.int32(1)), 6)

                @pl.when(j >= NBUF)
                def _():
                    pltpu.make_async_copy(
                        tbuf.at[b], out_hbm.at[0, :, pl.ds(0, RPB)], ssem
                    ).wait()

                @pl.loop(0, d, unroll=4)
                def _(di):
                    for k in range(RPB // 16):
                        rows = jax.lax.iota(jnp.int32, 16) + jnp.int32(16 * k)
                        cols = off_v[k, :] + di
                        vals = plsc.load_gather(gbuf.at[b], [rows, cols])
                        tbuf[b, di, pl.ds(16 * k, 16)] = vals

                pltpu.async_copy(
                    tbuf.at[b], out_hbm.at[j, :, pl.ds(b0, RPB)], ssem
                )
                @pl.when(j + NBUF < seq)
                def _():
                    pltpu.async_copy(
                        table_hbm.at[idx2_v.at[j + NBUF]], gbuf.at[b], gsem
                    )

        for b in range(NBUF):
            pltpu.make_async_copy(
                tbuf.at[b], out_hbm.at[0, :, pl.ds(0, RPB)], ssem
            ).wait()

    return gather_kernel


def kernel(input_ids, table):
    batch, seq = input_ids.shape
    v, d = table.shape
    assert batch == NW * RPB and seq % NBUF == 0 and v % 2 == 0
    ids_t = input_ids.T
    table2 = table.reshape(v // 2, 2 * d)
    out_t = _gather_fn(batch, seq, d)(ids_t, table2)
    return jnp.transpose(out_t, (2, 0, 1))


# speedup vs baseline: 1.1778x; 1.1778x over previous
import functools

import jax
import jax.numpy as jnp
from jax import lax
from jax.experimental import pallas as pl
from jax.experimental.pallas import tpu as pltpu
from jax.experimental.pallas import tpu_sc as plsc

NC = 2
NS = 16
NW = NC * NS
RPB = 128
NBUF = 2


def _gather_fn(batch, seq, d):
    mesh = plsc.VectorSubcoreMesh(
        core_axis_name="c", subcore_axis_name="s",
        num_cores=NC, num_subcores=NS,
    )
    dd = 2 * d

    @functools.partial(
        pl.kernel,
        out_type=jax.ShapeDtypeStruct((seq, d, batch), jnp.float32),
        mesh=mesh,
        compiler_params=pltpu.CompilerParams(needs_layout_passes=False),
        scratch_types=[
            pltpu.VMEM((seq, RPB), jnp.int32),
            pltpu.VMEM((seq, RPB), jnp.int32),
            pltpu.VMEM((8, 16), jnp.int32),
            pltpu.VMEM((NBUF, RPB, dd), jnp.float32),
            pltpu.VMEM((NBUF, d, RPB), jnp.float32),
            pltpu.SemaphoreType.DMA,
            pltpu.SemaphoreType.DMA,
        ],
    )
    def gather_kernel(ids_hbm, table_hbm, out_hbm, idx_v, idx2_v, off_v,
                      gbuf, tbuf, gsem, ssem):
        wid = lax.axis_index("s") * NC + lax.axis_index("c")
        b0 = wid * RPB
        pltpu.sync_copy(ids_hbm.at[:, pl.ds(b0, RPB)], idx_v)

        @pl.loop(0, seq)
        def _(j):
            for k in range(RPB // 16):
                v = idx_v[j, pl.ds(16 * k, 16)]
                idx2_v[j, pl.ds(16 * k, 16)] = lax.shift_right_logical(v, 1)

        for b in range(NBUF):
            pltpu.async_copy(table_hbm.at[idx2_v.at[b]], gbuf.at[b], gsem)

        @pl.loop(0, seq, step=NBUF)
        def _(g):
            for b in range(NBUF):
                j = g + b
                pltpu.make_async_copy(
                    table_hbm.at[idx2_v.at[0]], gbuf.at[b], gsem
                ).wait()
                @pl.when(j >= NBUF)
                def _():
                    pltpu.make_async_copy(
                        tbuf.at[b], out_hbm.at[0, :, pl.ds(0, RPB)], ssem
                    ).wait()

                for k in range(RPB // 16):
                    v = idx_v[j, pl.ds(16 * k, 16)]
                    off = lax.shift_left(lax.bitwise_and(v, jnp.int32(1)), 6)
                    rows = jax.lax.iota(jnp.int32, 16) + jnp.int32(16 * k)

                    @pl.loop(0, d, unroll=16)
                    def _(di):
                        vals = plsc.load_gather(gbuf.at[b], [rows, off + di])
                        tbuf[b, di, pl.ds(16 * k, 16)] = vals

                pltpu.async_copy(
                    tbuf.at[b], out_hbm.at[j, :, pl.ds(b0, RPB)], ssem
                )
                @pl.when(j + NBUF < seq)
                def _():
                    pltpu.async_copy(
                        table_hbm.at[idx2_v.at[j + NBUF]], gbuf.at[b], gsem
                    )

        for b in range(NBUF):
            pltpu.make_async_copy(
                tbuf.at[b], out_hbm.at[0, :, pl.ds(0, RPB)], ssem
            ).wait()

    return gather_kernel


def kernel(input_ids, table):
    batch, seq = input_ids.shape
    v, d = table.shape
    assert batch == NW * RPB and seq % NBUF == 0 and v % 2 == 0
    ids_t = input_ids.T
    table2 = table.reshape(v // 2, 2 * d)
    out_t = _gather_fn(batch, seq, d)(ids_t, table2)
    return jnp.transpose(out_t, (2, 0, 1))


# speedup vs baseline: 1.3996x; 1.1883x over previous
import functools

import jax
import jax.numpy as jnp
from jax import lax
from jax.experimental import pallas as pl
from jax.experimental.pallas import tpu as pltpu
from jax.experimental.pallas import tpu_sc as plsc

NC = 2
NS = 16
NW = NC * NS
RPB = 128
NBUF = 2


def _gather_fn(batch, seq, d):
    mesh = plsc.VectorSubcoreMesh(
        core_axis_name="c", subcore_axis_name="s",
        num_cores=NC, num_subcores=NS,
    )
    dd = 2 * d

    @functools.partial(
        pl.kernel,
        out_type=jax.ShapeDtypeStruct((seq, d, batch), jnp.float32),
        mesh=mesh,
        compiler_params=pltpu.CompilerParams(needs_layout_passes=False),
        scratch_types=[
            pltpu.VMEM((seq, RPB), jnp.int32),
            pltpu.VMEM((seq, RPB), jnp.int32),
            pltpu.VMEM((8, 16), jnp.int32),
            pltpu.VMEM((NBUF, RPB, dd), jnp.float32),
            pltpu.VMEM((NBUF, d, RPB), jnp.float32),
            pltpu.SemaphoreType.DMA,
            pltpu.SemaphoreType.DMA,
        ],
    )
    def gather_kernel(ids_hbm, table_hbm, out_hbm, idx_v, idx2_v, off_v,
                      gbuf, tbuf, gsem, ssem):
        wid = lax.axis_index("s") * NC + lax.axis_index("c")
        b0 = wid * RPB
        pltpu.sync_copy(ids_hbm.at[:, pl.ds(b0, RPB)], idx_v)

        @pl.loop(0, seq)
        def _(j):
            for k in range(RPB // 16):
                v = idx_v[j, pl.ds(16 * k, 16)]
                idx2_v[j, pl.ds(16 * k, 16)] = lax.shift_right_logical(v, 1)

        for b in range(NBUF):
            pltpu.async_copy(table_hbm.at[idx2_v.at[b]], gbuf.at[b], gsem)

        @pl.loop(0, seq, step=NBUF)
        def _(g):
            for b in range(NBUF):
                j = g + b
                pltpu.make_async_copy(
                    table_hbm.at[idx2_v.at[0]], gbuf.at[b], gsem
                ).wait()
                @pl.when(j >= NBUF)
                def _():
                    pltpu.make_async_copy(
                        tbuf.at[b], out_hbm.at[0, :, pl.ds(0, RPB)], ssem
                    ).wait()

                for k in range(RPB // 16):
                    v = idx_v[j, pl.ds(16 * k, 16)]
                    off = lax.shift_left(lax.bitwise_and(v, jnp.int32(1)), 6)
                    rows = jax.lax.iota(jnp.int32, 16) + jnp.int32(16 * k)

                    @plsc.parallel_loop(0, d, unroll=16)
                    def _(di):
                        vals = plsc.load_gather(gbuf.at[b], [rows, off + di])
                        tbuf[b, di, pl.ds(16 * k, 16)] = vals

                pltpu.async_copy(
                    tbuf.at[b], out_hbm.at[j, :, pl.ds(b0, RPB)], ssem
                )
                @pl.when(j + NBUF < seq)
                def _():
                    pltpu.async_copy(
                        table_hbm.at[idx2_v.at[j + NBUF]], gbuf.at[b], gsem
                    )

        for b in range(NBUF):
            pltpu.make_async_copy(
                tbuf.at[b], out_hbm.at[0, :, pl.ds(0, RPB)], ssem
            ).wait()

    return gather_kernel


def kernel(input_ids, table):
    batch, seq = input_ids.shape
    v, d = table.shape
    assert batch == NW * RPB and seq % NBUF == 0 and v % 2 == 0
    ids_t = input_ids.T
    table2 = table.reshape(v // 2, 2 * d)
    out_t = _gather_fn(batch, seq, d)(ids_t, table2)
    return jnp.transpose(out_t, (2, 0, 1))
